# baseline (speedup 1.0000x reference)
import jax
import jax.numpy as jnp
from jax import lax
from jax.experimental import pallas as pl
from jax.experimental.pallas import tpu as pltpu
from jax.experimental.pallas import tpu_sc as plsc

N = 10000
NPAD = 10240
E = 320000
DF = 128
DOUT = 128
DEGL = 40
GROUP = 5
NGRP = N // GROUP
NC = 2
NS = 16
NW = NC * NS
EPW = E // NW
CHUNK = 80
NCHUNK = EPW // CHUNK
RPW = NPAD // NS
SCHUNK = 128
SPW = 3 * SCHUNK
NSEED = NW * SPW
SPS = NSEED // NS

_R = 2000
_GRID = N // _R
_EB = 6400
_NEB = E // _EB


def _agg_loop(x_hbm, src_hbm, dst_hbm, e0, src_v, dst_v, rows_v, sems,
              agg_sh):
    pltpu.sync_copy(src_hbm.at[pl.ds(e0, CHUNK)], src_v[0])
    pltpu.sync_copy(dst_hbm.at[pl.ds(e0, CHUNK)], dst_v[0])
    pltpu.async_copy(x_hbm.at[src_v[0]], rows_v[0], sems[0])

    def body(gg, carry):
        for b in range(2):
            g = gg * 2 + b
            nb = 1 - b
            nbase = e0 + (g + 1) * CHUNK
            pltpu.sync_copy(src_hbm.at[pl.ds(nbase, CHUNK)], src_v[nb])
            pltpu.sync_copy(dst_hbm.at[pl.ds(nbase, CHUNK)], dst_v[nb])
            pltpu.async_copy(x_hbm.at[src_v[nb]], rows_v[nb], sems[nb])
            pltpu.make_async_copy(x_hbm.at[src_v[b]], rows_v[b],
                                  sems[b]).wait()
            pltpu.sync_copy(rows_v[b], agg_sh.at[dst_v[b]], add=True)
        return carry

    lax.fori_loop(0, (NCHUNK - 1) // 2, body, 0)
    lb = (NCHUNK - 1) % 2
    pltpu.make_async_copy(x_hbm.at[src_v[lb]], rows_v[lb], sems[lb]).wait()
    pltpu.sync_copy(rows_v[lb], agg_sh.at[dst_v[lb]], add=True)


def _sc_agg1(x_hbm, src_hbm, dst_hbm, z128_hbm, agg_out,
             src0_v, src1_v, dst0_v, dst1_v, rows0_v, rows1_v,
             sem0, sem1, agg_sh):
    c = lax.axis_index("c")
    s = lax.axis_index("s")
    w = s * NC + c
    r0 = s * RPW
    pltpu.sync_copy(z128_hbm.at[pl.ds(r0, RPW)], agg_sh.at[pl.ds(r0, RPW)])
    plsc.subcore_barrier()
    _agg_loop(x_hbm, src_hbm, dst_hbm, w * EPW,
              (src0_v, src1_v), (dst0_v, dst1_v), (rows0_v, rows1_v),
              (sem0, sem1), agg_sh)
    plsc.subcore_barrier()
    pltpu.sync_copy(agg_sh.at[pl.ds(r0, RPW)],
                    agg_out.at[c, pl.ds(r0, RPW)])


def _sc_agg2_seed(y2_hbm, h_hbm, src_hbm, dst_hbm, z128_hbm, sp_hbm,
                  embh_out, emba_out,
                  src0_v, src1_v, dst0_v, dst1_v, rows0_v, rows1_v,
                  sem0, sem1, sidx_v, grow_v, agg_sh):
    c = lax.axis_index("c")
    s = lax.axis_index("s")
    w = s * NC + c
    r0 = s * RPW
    pltpu.sync_copy(z128_hbm.at[pl.ds(r0, RPW)], agg_sh.at[pl.ds(r0, RPW)])
    plsc.subcore_barrier()
    _agg_loop(y2_hbm, src_hbm, dst_hbm, w * EPW,
              (src0_v, src1_v), (dst0_v, dst1_v), (rows0_v, rows1_v),
              (sem0, sem1), agg_sh)
    plsc.subcore_barrier()

    def body_h(j, carry):
        base = w * SPW + j * SCHUNK
        pltpu.sync_copy(sp_hbm.at[pl.ds(base, SCHUNK)], sidx_v)
        pltpu.sync_copy(h_hbm.at[sidx_v], grow_v)
        pltpu.sync_copy(grow_v, embh_out.at[pl.ds(base, SCHUNK)])
        return carry

    lax.fori_loop(0, SPW // SCHUNK, body_h, 0)

    def body_a(j, carry):
        base = s * SPS + j * SCHUNK
        pltpu.sync_copy(sp_hbm.at[pl.ds(base, SCHUNK)], sidx_v)
        pltpu.sync_copy(agg_sh.at[sidx_v], grow_v)
        pltpu.sync_copy(grow_v, emba_out.at[c, pl.ds(base, SCHUNK)])
        return carry

    lax.fori_loop(0, SPS // SCHUNK, body_a, 0)


_sc_calls = None


def _build_sc_calls():
    global _sc_calls
    if _sc_calls is not None:
        return _sc_calls
    mesh = plsc.VectorSubcoreMesh(core_axis_name="c", subcore_axis_name="s")
    ring_scratch = [
        pltpu.VMEM((CHUNK,), jnp.int32),
        pltpu.VMEM((CHUNK,), jnp.int32),
        pltpu.VMEM((CHUNK,), jnp.int32),
        pltpu.VMEM((CHUNK,), jnp.int32),
        pltpu.VMEM((CHUNK, DF), jnp.float32),
        pltpu.VMEM((CHUNK, DF), jnp.float32),
        pltpu.SemaphoreType.DMA,
        pltpu.SemaphoreType.DMA,
        pltpu.VMEM_SHARED((NPAD, DF), jnp.float32),
    ]
    agg1_call = pl.kernel(
        _sc_agg1, mesh=mesh,
        out_type=[jax.ShapeDtypeStruct((NC, NPAD, DF), jnp.float32)],
        scratch_types=list(ring_scratch))
    agg2_call = pl.kernel(
        _sc_agg2_seed, mesh=mesh,
        out_type=[jax.ShapeDtypeStruct((NSEED, DOUT), jnp.float32),
                  jax.ShapeDtypeStruct((NC, NSEED, DOUT), jnp.float32)],
        scratch_types=list(ring_scratch[:-1])
        + [pltpu.VMEM((SCHUNK,), jnp.int32),
           pltpu.VMEM((SCHUNK, DOUT), jnp.float32),
           ring_scratch[-1]])
    _sc_calls = (agg1_call, agg2_call)
    return _sc_calls


def _tc_deg(dst_ref, deg_ref):
    @pl.when(pl.program_id(0) == 0)
    def _():
        deg_ref[...] = jnp.zeros((128, 128), jnp.float32)

    d = dst_ref[...]
    lane = lax.broadcasted_iota(jnp.int32, (_EB, 128), 1)
    oh_hi = ((d >> 7) == lane).astype(jnp.float32)
    oh_lo = ((d & 127) == lane).astype(jnp.float32)
    deg_ref[...] += lax.dot_general(
        oh_hi, oh_lo, (((0,), (0,)), ((), ())),
        preferred_element_type=jnp.float32)


def _tc_deg_call(dst2):
    return pl.pallas_call(
        _tc_deg,
        grid=(_NEB,),
        in_specs=[pl.BlockSpec((_EB, 1), lambda i: (i, 0))],
        out_specs=pl.BlockSpec((128, 128), lambda i: (0, 0)),
        out_shape=jax.ShapeDtypeStruct((128, 128), jnp.float32),
    )(dst2)


def _tc_layer1(x_ref, agga_ref, aggb_ref, deg_ref,
               w_ref, wn_ref, wn2_ref, h_ref, y2_ref):
    deg = jnp.maximum(deg_ref[...], 1.0)
    agg = (agga_ref[0] + aggb_ref[0]) / deg
    h = jnp.maximum(
        jnp.dot(x_ref[...], w_ref[...], preferred_element_type=jnp.float32)
        + jnp.dot(agg, wn_ref[...], preferred_element_type=jnp.float32), 0.0)
    h_ref[...] = h
    lane = lax.broadcasted_iota(jnp.int32, (_R, DOUT), 1)
    y2_ref[...] = (jnp.dot(h, wn2_ref[...],
                           preferred_element_type=jnp.float32)
                   + (lane == DEGL).astype(jnp.float32))


def _tc_layer1_call(x, aggp, deg, w, wn, wn2):
    return pl.pallas_call(
        _tc_layer1,
        grid=(_GRID,),
        in_specs=[
            pl.BlockSpec((_R, DF), lambda i: (i, 0)),
            pl.BlockSpec((1, _R, DF), lambda i: (0, i, 0)),
            pl.BlockSpec((1, _R, DF), lambda i: (1, i, 0)),
            pl.BlockSpec((_R, 1), lambda i: (i, 0)),
            pl.BlockSpec((DF, DOUT), lambda i: (0, 0)),
            pl.BlockSpec((DF, DOUT), lambda i: (0, 0)),
            pl.BlockSpec((DOUT, DOUT), lambda i: (0, 0)),
        ],
        out_specs=[pl.BlockSpec((_R, DOUT), lambda i: (i, 0)),
                   pl.BlockSpec((_R, DOUT), lambda i: (i, 0))],
        out_shape=[jax.ShapeDtypeStruct((N, DOUT), jnp.float32),
                   jax.ShapeDtypeStruct((N, DOUT), jnp.float32)],
    )(x, aggp, aggp, deg, w, wn, wn2)


def _tc_loss(embh_ref, a0_ref, a1_ref, w2s_ref, out_ref):
    asum = a0_ref[0] + a1_ref[0]
    deg = jnp.maximum(asum[:, DEGL:DEGL + 1], 1.0)
    lane = lax.broadcasted_iota(jnp.int32, (NSEED, DOUT), 1)
    aggn = jnp.where(lane < DEGL, asum / deg, 0.0)
    emb = (jnp.dot(embh_ref[...], w2s_ref[...],
                   preferred_element_type=jnp.float32) + aggn)
    ctr = emb[0:NGRP]
    pos = emb[NGRP:2 * NGRP]
    n0 = emb[2 * NGRP:3 * NGRP]
    n1 = emb[3 * NGRP:4 * NGRP]
    n2 = emb[4 * NGRP:5 * NGRP]
    pos_d = jnp.sum(ctr * pos, axis=1, keepdims=True)
    d0 = jnp.sum(ctr * n0, axis=1, keepdims=True)
    d1 = jnp.sum(ctr * n1, axis=1, keepdims=True)
    d2 = jnp.sum(ctr * n2, axis=1, keepdims=True)
    neg_d = jnp.maximum(jnp.maximum(d0, d1), d2)
    loss = jnp.sum(jnp.maximum(neg_d - pos_d + 1.0, 0.0)) * (1.0 / NGRP)
    out_ref[...] = jnp.reshape(loss, (1, 1))


def _tc_loss_call(embh, emba, w2s):
    return pl.pallas_call(
        _tc_loss,
        grid=(1,),
        in_specs=[
            pl.BlockSpec((NSEED, DOUT), lambda i: (0, 0)),
            pl.BlockSpec((1, NSEED, DOUT), lambda i: (0, 0, 0)),
            pl.BlockSpec((1, NSEED, DOUT), lambda i: (1, 0, 0)),
            pl.BlockSpec((DOUT, DOUT), lambda i: (0, 0)),
        ],
        out_specs=pl.BlockSpec((1, 1), lambda i: (0, 0)),
        out_shape=jax.ShapeDtypeStruct((1, 1), jnp.float32),
    )(embh, emba, emba, w2s)


def kernel(x, edge_index, seed_idx, batch_idx, Wp1s, Wp1n, Wp2s, Wp2n,
           Wt1s, Wt1n, Wt2s, Wt2n):
    f32 = jnp.float32
    agg1_call, agg2_call = _build_sc_calls()
    src = edge_index[0]
    dst = edge_index[1]
    z128 = jnp.zeros((NPAD, DF), f32)
    (agg1p,) = agg1_call(x, src, dst, z128)

    deg2d = _tc_deg_call(dst.reshape(E, 1))
    deg = deg2d.reshape(128 * 128, 1)[0:N]

    pad = jnp.zeros((DF, DOUT - 40), f32)
    W1 = jnp.concatenate([Wp1s, Wt1s, pad], axis=1)
    W1n = jnp.concatenate([Wp1n, Wt1n, pad], axis=1)
    W2n_blk = jnp.zeros((DOUT, DOUT), f32)
    W2n_blk = W2n_blk.at[0:32, 0:32].set(Wp2n).at[32:40, 32:40].set(Wt2n)
    h, y2 = _tc_layer1_call(x, agg1p, deg, W1, W1n, W2n_blk)

    sp = seed_idx.reshape(NGRP, GROUP).T.reshape(-1)
    sp = jnp.concatenate([sp, jnp.zeros((NSEED - N,), jnp.int32)])
    embh, emba = agg2_call(y2, h, src, dst, z128, sp)

    W2s_blk = jnp.zeros((DOUT, DOUT), f32)
    W2s_blk = W2s_blk.at[0:32, 0:32].set(Wp2s).at[32:40, 32:40].set(Wt2s)
    loss = _tc_loss_call(embh, emba, W2s_blk)
    return loss[0, 0]
